# baseline (speedup 1.0000x reference)
import functools

import jax
import jax.numpy as jnp
from jax import lax
from jax.experimental import pallas as pl
from jax.experimental.pallas import tpu as pltpu
from jax.experimental.pallas import tpu_sc as plsc

N_NODES = 10000
N_EDGES = 320000
D_IN = 128
D_OUT = 128
N_REL = 50
N_BASES = 30

NC = 2
NS = 16
NW = NC * NS
CHUNK = 128
E_PAD = 327680
EPT = E_PAD // NW
NCHUNK = EPT // CHUNK
N_PAD = 10112
NROW_PT = N_PAD // NS


def _wfull_body(wcoe_ref, weight_ref, out_ref):
    out_ref[...] = jnp.dot(wcoe_ref[...], weight_ref[...],
                           preferred_element_type=jnp.float32)


def _make_wfull(w_coe, weight):
    weight2 = weight.reshape(N_BASES, D_IN * D_OUT)
    out = pl.pallas_call(
        _wfull_body,
        out_shape=jax.ShapeDtypeStruct((N_REL, D_IN * D_OUT), jnp.float32),
    )(w_coe, weight2)
    return out.reshape(N_REL, D_IN, D_OUT)


_BN = 1000


def _trans_body(x_ref, wf_ref, out_ref):
    out_ref[0] = jnp.dot(x_ref[...], wf_ref[0],
                         preferred_element_type=jnp.float32)


def _make_trans(x, w_full):
    grid = (N_NODES // _BN, N_REL)
    return pl.pallas_call(
        _trans_body,
        grid=grid,
        in_specs=[
            pl.BlockSpec((_BN, D_IN), lambda i, j: (i, 0)),
            pl.BlockSpec((1, D_IN, D_OUT), lambda i, j: (j, 0, 0)),
        ],
        out_specs=pl.BlockSpec((1, _BN, D_OUT), lambda i, j: (j, i, 0)),
        out_shape=jax.ShapeDtypeStruct((N_REL, N_NODES, D_OUT), jnp.float32),
    )(x, w_full)


def _idx_body(src_ref, et_ref, o_ref):
    o_ref[...] = et_ref[...] * N_NODES + src_ref[...]


def _make_idx(src_p, et_p):
    nrow = E_PAD // CHUNK
    blk = 256
    return pl.pallas_call(
        _idx_body,
        grid=(nrow // blk,),
        in_specs=[pl.BlockSpec((blk, CHUNK), lambda i: (i, 0)),
                  pl.BlockSpec((blk, CHUNK), lambda i: (i, 0))],
        out_specs=pl.BlockSpec((blk, CHUNK), lambda i: (i, 0)),
        out_shape=jax.ShapeDtypeStruct((nrow, CHUNK), jnp.int32),
    )(src_p.reshape(nrow, CHUNK), et_p.reshape(nrow, CHUNK))


def _sc_body(trans_hbm, idx_hbm, dst_hbm, norm_hbm, zero_hbm,
             out_hbm, idx_v, dst_v, norm_v, rows_v, agg_sh, gsem, gsem2):
    c = lax.axis_index("c")
    s = lax.axis_index("s")
    w = c * NS + s

    zr0 = s * NROW_PT
    pltpu.sync_copy(zero_hbm.at[pl.ds(zr0, NROW_PT)],
                    agg_sh.at[pl.ds(zr0, NROW_PT)])

    plsc.subcore_barrier()

    @pl.when(c == 1)
    def _work():
        def half(h, _):
            w2 = h * NS + s
            pltpu.sync_copy(idx_hbm.at[w2], idx_v)
            pltpu.sync_copy(dst_hbm.at[w2], dst_v)
            pltpu.sync_copy(norm_hbm.at[w2], norm_v)

            def _chunk(j, _):
                d1 = pltpu.async_copy(trans_hbm.at[idx_v.at[j, pl.ds(0, 64)]],
                                      rows_v.at[pl.ds(0, 64)], gsem)
                d2 = pltpu.async_copy(trans_hbm.at[idx_v.at[j, pl.ds(64, 64)]],
                                      rows_v.at[pl.ds(64, 64)], gsem2)
                d1.wait()
                d2.wait()

                def grp(g, _):
                    nvec = norm_v[j, pl.ds(g * 16, 16)]
                    row0 = g * 16
                    for e in range(16):
                        nsp = jnp.take(nvec, jnp.full((16,), e, jnp.int32))
                        for k in range(D_OUT // 16):
                            sl = pl.ds(k * 16, 16)
                            rows_v[row0 + e, sl] = rows_v[row0 + e, sl] * nsp
                    return 0
                lax.fori_loop(0, CHUNK // 16, grp, 0)

                pltpu.sync_copy(rows_v, agg_sh.at[dst_v.at[j]], add=True)
                return 0
            lax.fori_loop(0, NCHUNK, _chunk, 0)
            return 0
        lax.fori_loop(0, NC, half, 0)

    plsc.subcore_barrier()

    pltpu.sync_copy(agg_sh.at[pl.ds(zr0, NROW_PT)],
                    out_hbm.at[c, pl.ds(zr0, NROW_PT)])


def _sc_scatter(trans2, idx2, dst2, norm2, zeros):
    mesh = plsc.VectorSubcoreMesh(core_axis_name="c", subcore_axis_name="s",
                                  num_cores=NC, num_subcores=NS)
    f = pl.kernel(
        _sc_body,
        out_type=jax.ShapeDtypeStruct((NC, N_PAD, D_OUT), jnp.float32),
        mesh=mesh,
        scratch_types=[
            pltpu.VMEM((NCHUNK, CHUNK), jnp.int32),
            pltpu.VMEM((NCHUNK, CHUNK), jnp.int32),
            pltpu.VMEM((NCHUNK, CHUNK), jnp.float32),
            pltpu.VMEM((CHUNK, D_OUT), jnp.float32),
            pltpu.VMEM_SHARED((N_PAD, D_OUT), jnp.float32),
            pltpu.SemaphoreType.DMA,
            pltpu.SemaphoreType.DMA,
        ],
    )
    return f(trans2, idx2, dst2, norm2, zeros)


def _final_body(p_ref, x_ref, sl_ref, o_ref):
    o_ref[...] = (p_ref[0] + p_ref[1] +
                  jnp.dot(x_ref[...], sl_ref[...],
                          preferred_element_type=jnp.float32))


def _final(parts, x, self_loop):
    grid = (N_NODES // _BN,)
    return pl.pallas_call(
        _final_body,
        grid=grid,
        in_specs=[
            pl.BlockSpec((NC, _BN, D_OUT), lambda i: (0, i, 0)),
            pl.BlockSpec((_BN, D_IN), lambda i: (i, 0)),
            pl.BlockSpec((D_IN, D_OUT), lambda i: (0, 0)),
        ],
        out_specs=pl.BlockSpec((_BN, D_OUT), lambda i: (i, 0)),
        out_shape=jax.ShapeDtypeStruct((N_NODES, D_OUT), jnp.float32),
    )(parts, x, self_loop)


def kernel(x, edge_index, edge_type, norm, weight, w_coe, self_loop):
    w_full = _make_wfull(w_coe, weight)
    trans = _make_trans(x, w_full)
    trans2 = trans.reshape(N_REL * N_NODES, D_OUT)

    pad = E_PAD - N_EDGES
    src_p = jnp.concatenate([edge_index[0], jnp.zeros((pad,), jnp.int32)])
    dst_p = jnp.concatenate([edge_index[1],
                             jnp.full((pad,), N_NODES, jnp.int32)])
    et_p = jnp.concatenate([edge_type, jnp.zeros((pad,), jnp.int32)])
    norm_p = jnp.concatenate([norm.reshape(N_EDGES),
                              jnp.zeros((pad,), jnp.float32)])

    idx2 = _make_idx(src_p, et_p).reshape(NW, NCHUNK, CHUNK)
    dst2 = dst_p.reshape(NW, NCHUNK, CHUNK)
    norm2 = norm_p.reshape(NW, NCHUNK, CHUNK)
    zeros = jnp.zeros((N_PAD, D_OUT), jnp.float32)

    parts = _sc_scatter(trans2, idx2, dst2, norm2, zeros)
    return _final(parts[:, :N_NODES], x, self_loop)


# speedup vs baseline: 1.1925x; 1.1925x over previous
import functools

import jax
import jax.numpy as jnp
from jax import lax
from jax.experimental import pallas as pl
from jax.experimental.pallas import tpu as pltpu
from jax.experimental.pallas import tpu_sc as plsc

N_NODES = 10000
N_EDGES = 320000
D_IN = 128
D_OUT = 128
N_REL = 50
N_BASES = 30

NC = 2
NS = 16
NW = NC * NS
CHUNK = 128
E_PAD = 327680
EPT = E_PAD // NW
NCHUNK = EPT // CHUNK
N_PAD = 10112
NROW_PT = N_PAD // NS


def _wfull_body(wcoe_ref, weight_ref, out_ref):
    out_ref[...] = jnp.dot(wcoe_ref[...], weight_ref[...],
                           preferred_element_type=jnp.float32)


def _make_wfull(w_coe, weight):
    weight2 = weight.reshape(N_BASES, D_IN * D_OUT)
    out = pl.pallas_call(
        _wfull_body,
        out_shape=jax.ShapeDtypeStruct((N_REL, D_IN * D_OUT), jnp.float32),
    )(w_coe, weight2)
    return out.reshape(N_REL, D_IN, D_OUT)


_BN = 1000


def _trans_body(x_ref, wf_ref, out_ref):
    out_ref[0] = jnp.dot(x_ref[...], wf_ref[0],
                         preferred_element_type=jnp.float32)


def _make_trans(x, w_full):
    grid = (N_NODES // _BN, N_REL)
    return pl.pallas_call(
        _trans_body,
        grid=grid,
        in_specs=[
            pl.BlockSpec((_BN, D_IN), lambda i, j: (i, 0)),
            pl.BlockSpec((1, D_IN, D_OUT), lambda i, j: (j, 0, 0)),
        ],
        out_specs=pl.BlockSpec((1, _BN, D_OUT), lambda i, j: (j, i, 0)),
        out_shape=jax.ShapeDtypeStruct((N_REL, N_NODES, D_OUT), jnp.float32),
    )(x, w_full)


def _idx_body(src_ref, et_ref, o_ref):
    o_ref[...] = et_ref[...] * N_NODES + src_ref[...]


def _make_idx(src_p, et_p):
    nrow = E_PAD // CHUNK
    blk = 256
    return pl.pallas_call(
        _idx_body,
        grid=(nrow // blk,),
        in_specs=[pl.BlockSpec((blk, CHUNK), lambda i: (i, 0)),
                  pl.BlockSpec((blk, CHUNK), lambda i: (i, 0))],
        out_specs=pl.BlockSpec((blk, CHUNK), lambda i: (i, 0)),
        out_shape=jax.ShapeDtypeStruct((nrow, CHUNK), jnp.int32),
    )(src_p.reshape(nrow, CHUNK), et_p.reshape(nrow, CHUNK))


def _sc_body(trans_hbm, idx_hbm, dst_hbm, norm_hbm, zero_hbm,
             out_hbm, idx_v, dst_v, norm_v, rows_v, agg_sh, gsem, gsem2):
    c = lax.axis_index("c")
    s = lax.axis_index("s")
    w = c * NS + s

    zr0 = s * NROW_PT
    pltpu.sync_copy(zero_hbm.at[pl.ds(zr0, NROW_PT)],
                    agg_sh.at[pl.ds(zr0, NROW_PT)])

    pltpu.sync_copy(idx_hbm.at[w], idx_v)
    pltpu.sync_copy(dst_hbm.at[w], dst_v)
    pltpu.sync_copy(norm_hbm.at[w], norm_v)

    plsc.subcore_barrier()

    def scale_half(j, half, lo):
        def grp(g, _):
            nvec = norm_v[j, pl.ds(lo + g * 16, 16)]
            row0 = lo + g * 16
            for e in range(16):
                nsp = jnp.take(nvec, jnp.full((16,), e, jnp.int32))
                for k in range(D_OUT // 16):
                    sl = pl.ds(k * 16, 16)
                    rows_v[row0 + e, sl] = rows_v[row0 + e, sl] * nsp
            return 0
        lax.fori_loop(0, CHUNK // 32, grp, 0)

    def _chunk(j, _):
        d1 = pltpu.async_copy(trans_hbm.at[idx_v.at[j, pl.ds(0, 64)]],
                              rows_v.at[pl.ds(0, 64)], gsem)
        d2 = pltpu.async_copy(trans_hbm.at[idx_v.at[j, pl.ds(64, 64)]],
                              rows_v.at[pl.ds(64, 64)], gsem2)
        d1.wait()
        scale_half(j, 0, 0)
        d2.wait()
        scale_half(j, 1, 64)
        pltpu.sync_copy(rows_v, agg_sh.at[dst_v.at[j]], add=True)
        return 0
    lax.fori_loop(0, NCHUNK, _chunk, 0)

    plsc.subcore_barrier()

    pltpu.sync_copy(agg_sh.at[pl.ds(zr0, NROW_PT)],
                    out_hbm.at[c, pl.ds(zr0, NROW_PT)])


def _sc_scatter(trans2, idx2, dst2, norm2, zeros):
    mesh = plsc.VectorSubcoreMesh(core_axis_name="c", subcore_axis_name="s",
                                  num_cores=NC, num_subcores=NS)
    f = pl.kernel(
        _sc_body,
        out_type=jax.ShapeDtypeStruct((NC, N_PAD, D_OUT), jnp.float32),
        mesh=mesh,
        scratch_types=[
            pltpu.VMEM((NCHUNK, CHUNK), jnp.int32),
            pltpu.VMEM((NCHUNK, CHUNK), jnp.int32),
            pltpu.VMEM((NCHUNK, CHUNK), jnp.float32),
            pltpu.VMEM((CHUNK, D_OUT), jnp.float32),
            pltpu.VMEM_SHARED((N_PAD, D_OUT), jnp.float32),
            pltpu.SemaphoreType.DMA,
            pltpu.SemaphoreType.DMA,
        ],
    )
    return f(trans2, idx2, dst2, norm2, zeros)


def _final_body(p_ref, x_ref, sl_ref, o_ref):
    o_ref[...] = (p_ref[0] + p_ref[1] +
                  jnp.dot(x_ref[...], sl_ref[...],
                          preferred_element_type=jnp.float32))


def _final(parts, x, self_loop):
    grid = (N_NODES // _BN,)
    return pl.pallas_call(
        _final_body,
        grid=grid,
        in_specs=[
            pl.BlockSpec((NC, _BN, D_OUT), lambda i: (0, i, 0)),
            pl.BlockSpec((_BN, D_IN), lambda i: (i, 0)),
            pl.BlockSpec((D_IN, D_OUT), lambda i: (0, 0)),
        ],
        out_specs=pl.BlockSpec((_BN, D_OUT), lambda i: (i, 0)),
        out_shape=jax.ShapeDtypeStruct((N_NODES, D_OUT), jnp.float32),
    )(parts, x, self_loop)


def kernel(x, edge_index, edge_type, norm, weight, w_coe, self_loop):
    w_full = _make_wfull(w_coe, weight)
    trans = _make_trans(x, w_full)
    trans2 = trans.reshape(N_REL * N_NODES, D_OUT)

    pad = E_PAD - N_EDGES
    src_p = jnp.concatenate([edge_index[0], jnp.zeros((pad,), jnp.int32)])
    dst_p = jnp.concatenate([edge_index[1],
                             jnp.full((pad,), N_NODES, jnp.int32)])
    et_p = jnp.concatenate([edge_type, jnp.zeros((pad,), jnp.int32)])
    norm_p = jnp.concatenate([norm.reshape(N_EDGES),
                              jnp.zeros((pad,), jnp.float32)])

    idx2 = _make_idx(src_p, et_p).reshape(NW, NCHUNK, CHUNK)
    dst2 = dst_p.reshape(NW, NCHUNK, CHUNK)
    norm2 = norm_p.reshape(NW, NCHUNK, CHUNK)
    zeros = jnp.zeros((N_PAD, D_OUT), jnp.float32)

    parts = _sc_scatter(trans2, idx2, dst2, norm2, zeros)
    return _final(parts[:, :N_NODES], x, self_loop)


# speedup vs baseline: 1.4233x; 1.1935x over previous
import functools

import jax
import jax.numpy as jnp
from jax import lax
from jax.experimental import pallas as pl
from jax.experimental.pallas import tpu as pltpu
from jax.experimental.pallas import tpu_sc as plsc

N_NODES = 10000
N_EDGES = 320000
D_IN = 128
D_OUT = 128
N_REL = 50
N_BASES = 30

NC = 2
NS = 16
NW = NC * NS
CHUNK = 128
E_PAD = 327680
EPT = E_PAD // NW
NCHUNK = EPT // CHUNK
N_PAD = 10112
NROW_PT = N_PAD // NS


def _wfull_body(wcoe_ref, weight_ref, out_ref):
    out_ref[...] = jnp.dot(wcoe_ref[...], weight_ref[...],
                           preferred_element_type=jnp.float32)


def _make_wfull(w_coe, weight):
    weight2 = weight.reshape(N_BASES, D_IN * D_OUT)
    out = pl.pallas_call(
        _wfull_body,
        out_shape=jax.ShapeDtypeStruct((N_REL, D_IN * D_OUT), jnp.float32),
    )(w_coe, weight2)
    return out.reshape(N_REL, D_IN, D_OUT)


_BN = 2000


def _trans_body(x_ref, wf_ref, out_ref):
    out_ref[0] = jnp.dot(x_ref[...], wf_ref[0],
                         preferred_element_type=jnp.float32)


def _make_trans(x, w_full):
    grid = (N_NODES // _BN, N_REL)
    return pl.pallas_call(
        _trans_body,
        grid=grid,
        in_specs=[
            pl.BlockSpec((_BN, D_IN), lambda i, j: (i, 0)),
            pl.BlockSpec((1, D_IN, D_OUT), lambda i, j: (j, 0, 0)),
        ],
        out_specs=pl.BlockSpec((1, _BN, D_OUT), lambda i, j: (j, i, 0)),
        out_shape=jax.ShapeDtypeStruct((N_REL, N_NODES, D_OUT), jnp.float32),
    )(x, w_full)


def _idx_body(src_ref, et_ref, o_ref):
    o_ref[...] = et_ref[...] * N_NODES + src_ref[...]


def _make_idx(src_p, et_p):
    nrow = E_PAD // CHUNK
    blk = 256
    return pl.pallas_call(
        _idx_body,
        grid=(nrow // blk,),
        in_specs=[pl.BlockSpec((blk, CHUNK), lambda i: (i, 0)),
                  pl.BlockSpec((blk, CHUNK), lambda i: (i, 0))],
        out_specs=pl.BlockSpec((blk, CHUNK), lambda i: (i, 0)),
        out_shape=jax.ShapeDtypeStruct((nrow, CHUNK), jnp.int32),
    )(src_p.reshape(nrow, CHUNK), et_p.reshape(nrow, CHUNK))


def _sc_body(trans_hbm, idx_hbm, dst_hbm, norm_hbm, zero_hbm,
             out_hbm, idx_v, dst_v, norm_v, rows_v, agg_sh, gsem, gsem2):
    c = lax.axis_index("c")
    s = lax.axis_index("s")
    w = c * NS + s

    zr0 = s * NROW_PT
    pltpu.sync_copy(zero_hbm.at[pl.ds(zr0, NROW_PT)],
                    agg_sh.at[pl.ds(zr0, NROW_PT)])

    pltpu.sync_copy(idx_hbm.at[w], idx_v)
    pltpu.sync_copy(dst_hbm.at[w], dst_v)
    pltpu.sync_copy(norm_hbm.at[w], norm_v)

    plsc.subcore_barrier()

    def scale_half(j, half, lo):
        def grp(g, _):
            nvec = norm_v[j, pl.ds(lo + g * 16, 16)]
            row0 = lo + g * 16
            for e in range(16):
                nsp = jnp.take(nvec, jnp.full((16,), e, jnp.int32))
                for k in range(D_OUT // 16):
                    sl = pl.ds(k * 16, 16)
                    rows_v[row0 + e, sl] = rows_v[row0 + e, sl] * nsp
            return 0
        lax.fori_loop(0, CHUNK // 32, grp, 0)

    def _chunk(j, _):
        d1 = pltpu.async_copy(trans_hbm.at[idx_v.at[j, pl.ds(0, 64)]],
                              rows_v.at[pl.ds(0, 64)], gsem)
        d2 = pltpu.async_copy(trans_hbm.at[idx_v.at[j, pl.ds(64, 64)]],
                              rows_v.at[pl.ds(64, 64)], gsem2)
        d1.wait()
        scale_half(j, 0, 0)
        d2.wait()
        scale_half(j, 1, 64)
        pltpu.sync_copy(rows_v, agg_sh.at[dst_v.at[j]], add=True)
        return 0
    lax.fori_loop(0, NCHUNK, _chunk, 0)

    plsc.subcore_barrier()

    pltpu.sync_copy(agg_sh.at[pl.ds(zr0, NROW_PT)],
                    out_hbm.at[c, pl.ds(zr0, NROW_PT)])


def _sc_scatter(trans2, idx2, dst2, norm2, zeros):
    mesh = plsc.VectorSubcoreMesh(core_axis_name="c", subcore_axis_name="s",
                                  num_cores=NC, num_subcores=NS)
    f = pl.kernel(
        _sc_body,
        out_type=jax.ShapeDtypeStruct((NC, N_PAD, D_OUT), jnp.float32),
        mesh=mesh,
        scratch_types=[
            pltpu.VMEM((NCHUNK, CHUNK), jnp.int32),
            pltpu.VMEM((NCHUNK, CHUNK), jnp.int32),
            pltpu.VMEM((NCHUNK, CHUNK), jnp.float32),
            pltpu.VMEM((CHUNK, D_OUT), jnp.float32),
            pltpu.VMEM_SHARED((N_PAD, D_OUT), jnp.float32),
            pltpu.SemaphoreType.DMA,
            pltpu.SemaphoreType.DMA,
        ],
    )
    return f(trans2, idx2, dst2, norm2, zeros)


def _final_body(p_ref, x_ref, sl_ref, o_ref):
    o_ref[...] = (p_ref[0] + p_ref[1] +
                  jnp.dot(x_ref[...], sl_ref[...],
                          preferred_element_type=jnp.float32))


def _final(parts, x, self_loop):
    grid = (N_NODES // _BN,)
    return pl.pallas_call(
        _final_body,
        grid=grid,
        in_specs=[
            pl.BlockSpec((NC, _BN, D_OUT), lambda i: (0, i, 0)),
            pl.BlockSpec((_BN, D_IN), lambda i: (i, 0)),
            pl.BlockSpec((D_IN, D_OUT), lambda i: (0, 0)),
        ],
        out_specs=pl.BlockSpec((_BN, D_OUT), lambda i: (i, 0)),
        out_shape=jax.ShapeDtypeStruct((N_NODES, D_OUT), jnp.float32),
    )(parts, x, self_loop)


def kernel(x, edge_index, edge_type, norm, weight, w_coe, self_loop):
    w_full = _make_wfull(w_coe, weight)
    trans = _make_trans(x, w_full)
    trans2 = trans.reshape(N_REL * N_NODES, D_OUT)

    pad = E_PAD - N_EDGES
    src_p = jnp.concatenate([edge_index[0], jnp.zeros((pad,), jnp.int32)])
    dst_p = jnp.concatenate([edge_index[1],
                             jnp.full((pad,), N_NODES, jnp.int32)])
    et_p = jnp.concatenate([edge_type, jnp.zeros((pad,), jnp.int32)])
    norm_p = jnp.concatenate([norm.reshape(N_EDGES),
                              jnp.zeros((pad,), jnp.float32)])

    idx2 = _make_idx(src_p, et_p).reshape(NW, NCHUNK, CHUNK)
    dst2 = dst_p.reshape(NW, NCHUNK, CHUNK)
    norm2 = norm_p.reshape(NW, NCHUNK, CHUNK)
    zeros = jnp.zeros((N_PAD, D_OUT), jnp.float32)

    parts = _sc_scatter(trans2, idx2, dst2, norm2, zeros)
    return _final(parts[:, :N_NODES], x, self_loop)


# speedup vs baseline: 1.6783x; 1.1791x over previous
import functools

import jax
import jax.numpy as jnp
from jax import lax
from jax.experimental import pallas as pl
from jax.experimental.pallas import tpu as pltpu
from jax.experimental.pallas import tpu_sc as plsc

N_NODES = 10000
N_EDGES = 320000
D_IN = 128
D_OUT = 128
N_REL = 50
N_BASES = 30

NC = 2
NS = 16
NW = NC * NS
CHUNK = 128
E_PAD = 327680
EPT = E_PAD // NW
NCHUNK = EPT // CHUNK
N_PAD = 10112
NROW_PT = N_PAD // NS


def _wfull_body(wcoe_ref, weight_ref, out_ref):
    out_ref[...] = jnp.dot(wcoe_ref[...], weight_ref[...],
                           preferred_element_type=jnp.float32)


def _make_wfull(w_coe, weight):
    weight2 = weight.reshape(N_BASES, D_IN * D_OUT)
    out = pl.pallas_call(
        _wfull_body,
        out_shape=jax.ShapeDtypeStruct((N_REL, D_IN * D_OUT), jnp.float32),
    )(w_coe, weight2)
    return out.reshape(N_REL, D_IN, D_OUT)


_BN = 10000


def _trans_body(x_ref, wf_ref, out_ref):
    out_ref[0] = jnp.dot(x_ref[...], wf_ref[0],
                         preferred_element_type=jnp.float32)


def _make_trans(x, w_full):
    grid = (N_NODES // _BN, N_REL)
    return pl.pallas_call(
        _trans_body,
        grid=grid,
        in_specs=[
            pl.BlockSpec((_BN, D_IN), lambda i, j: (i, 0)),
            pl.BlockSpec((1, D_IN, D_OUT), lambda i, j: (j, 0, 0)),
        ],
        out_specs=pl.BlockSpec((1, _BN, D_OUT), lambda i, j: (j, i, 0)),
        out_shape=jax.ShapeDtypeStruct((N_REL, N_NODES, D_OUT), jnp.float32),
    )(x, w_full)


def _idx_body(src_ref, et_ref, o_ref):
    o_ref[...] = et_ref[...] * N_NODES + src_ref[...]


def _make_idx(src_p, et_p):
    nrow = E_PAD // CHUNK
    blk = 256
    return pl.pallas_call(
        _idx_body,
        grid=(nrow // blk,),
        in_specs=[pl.BlockSpec((blk, CHUNK), lambda i: (i, 0)),
                  pl.BlockSpec((blk, CHUNK), lambda i: (i, 0))],
        out_specs=pl.BlockSpec((blk, CHUNK), lambda i: (i, 0)),
        out_shape=jax.ShapeDtypeStruct((nrow, CHUNK), jnp.int32),
    )(src_p.reshape(nrow, CHUNK), et_p.reshape(nrow, CHUNK))


def _sc_body(trans_hbm, idx_hbm, dst_hbm, norm_hbm, zero_hbm,
             out_hbm, idx_v, dst_v, norm_v, rows_v, agg_sh, gsem, gsem2):
    c = lax.axis_index("c")
    s = lax.axis_index("s")
    w = c * NS + s

    zr0 = s * NROW_PT
    pltpu.sync_copy(zero_hbm.at[pl.ds(zr0, NROW_PT)],
                    agg_sh.at[pl.ds(zr0, NROW_PT)])

    pltpu.sync_copy(idx_hbm.at[w], idx_v)
    pltpu.sync_copy(dst_hbm.at[w], dst_v)
    pltpu.sync_copy(norm_hbm.at[w], norm_v)

    plsc.subcore_barrier()

    def scale_half(j, half, lo):
        def grp(g, _):
            nvec = norm_v[j, pl.ds(lo + g * 16, 16)]
            row0 = lo + g * 16
            for e in range(16):
                nsp = jnp.take(nvec, jnp.full((16,), e, jnp.int32))
                for k in range(D_OUT // 16):
                    sl = pl.ds(k * 16, 16)
                    rows_v[row0 + e, sl] = rows_v[row0 + e, sl] * nsp
            return 0
        lax.fori_loop(0, CHUNK // 32, grp, 0)

    def _chunk(j, _):
        d1 = pltpu.async_copy(trans_hbm.at[idx_v.at[j, pl.ds(0, 64)]],
                              rows_v.at[pl.ds(0, 64)], gsem)
        d2 = pltpu.async_copy(trans_hbm.at[idx_v.at[j, pl.ds(64, 64)]],
                              rows_v.at[pl.ds(64, 64)], gsem2)
        d1.wait()
        scale_half(j, 0, 0)
        d2.wait()
        scale_half(j, 1, 64)
        pltpu.sync_copy(rows_v, agg_sh.at[dst_v.at[j]], add=True)
        return 0
    lax.fori_loop(0, NCHUNK, _chunk, 0)

    plsc.subcore_barrier()

    pltpu.sync_copy(agg_sh.at[pl.ds(zr0, NROW_PT)],
                    out_hbm.at[c, pl.ds(zr0, NROW_PT)])


def _sc_scatter(trans2, idx2, dst2, norm2, zeros):
    mesh = plsc.VectorSubcoreMesh(core_axis_name="c", subcore_axis_name="s",
                                  num_cores=NC, num_subcores=NS)
    f = pl.kernel(
        _sc_body,
        out_type=jax.ShapeDtypeStruct((NC, N_PAD, D_OUT), jnp.float32),
        mesh=mesh,
        scratch_types=[
            pltpu.VMEM((NCHUNK, CHUNK), jnp.int32),
            pltpu.VMEM((NCHUNK, CHUNK), jnp.int32),
            pltpu.VMEM((NCHUNK, CHUNK), jnp.float32),
            pltpu.VMEM((CHUNK, D_OUT), jnp.float32),
            pltpu.VMEM_SHARED((N_PAD, D_OUT), jnp.float32),
            pltpu.SemaphoreType.DMA,
            pltpu.SemaphoreType.DMA,
        ],
    )
    return f(trans2, idx2, dst2, norm2, zeros)


def _final_body(p_ref, x_ref, sl_ref, o_ref):
    o_ref[...] = (p_ref[0] + p_ref[1] +
                  jnp.dot(x_ref[...], sl_ref[...],
                          preferred_element_type=jnp.float32))


def _final(parts, x, self_loop):
    grid = (N_NODES // _BN,)
    return pl.pallas_call(
        _final_body,
        grid=grid,
        in_specs=[
            pl.BlockSpec((NC, _BN, D_OUT), lambda i: (0, i, 0)),
            pl.BlockSpec((_BN, D_IN), lambda i: (i, 0)),
            pl.BlockSpec((D_IN, D_OUT), lambda i: (0, 0)),
        ],
        out_specs=pl.BlockSpec((_BN, D_OUT), lambda i: (i, 0)),
        out_shape=jax.ShapeDtypeStruct((N_NODES, D_OUT), jnp.float32),
    )(parts, x, self_loop)


def kernel(x, edge_index, edge_type, norm, weight, w_coe, self_loop):
    w_full = _make_wfull(w_coe, weight)
    trans = _make_trans(x, w_full)
    trans2 = trans.reshape(N_REL * N_NODES, D_OUT)

    pad = E_PAD - N_EDGES
    src_p = jnp.concatenate([edge_index[0], jnp.zeros((pad,), jnp.int32)])
    dst_p = jnp.concatenate([edge_index[1],
                             jnp.full((pad,), N_NODES, jnp.int32)])
    et_p = jnp.concatenate([edge_type, jnp.zeros((pad,), jnp.int32)])
    norm_p = jnp.concatenate([norm.reshape(N_EDGES),
                              jnp.zeros((pad,), jnp.float32)])

    idx2 = _make_idx(src_p, et_p).reshape(NW, NCHUNK, CHUNK)
    dst2 = dst_p.reshape(NW, NCHUNK, CHUNK)
    norm2 = norm_p.reshape(NW, NCHUNK, CHUNK)
    zeros = jnp.zeros((N_PAD, D_OUT), jnp.float32)

    parts = _sc_scatter(trans2, idx2, dst2, norm2, zeros)
    return _final(parts[:, :N_NODES], x, self_loop)
